# baseline (speedup 1.0000x reference)
import jax
import jax.numpy as jnp
from jax.experimental import pallas as pl
from jax.experimental.pallas import tpu as pltpu

_N = 4
_CH = 512


def _body(x_hbm, pe_hbm, o_hbm, xbuf, pebuf, xsem, pesem, osem):
    B, S, D = x_hbm.shape
    NP = S // _CH
    NT = NP * B

    def start_xin(t):
        p, b = divmod(t, B)
        cp = pltpu.make_async_copy(
            x_hbm.at[b, pl.ds(p * _CH, _CH)], xbuf.at[t % _N],
            xsem.at[t % _N])
        cp.start()
        return cp

    def start_pin(p):
        cp = pltpu.make_async_copy(
            pe_hbm.at[pl.ds(p * _CH, _CH)], pebuf.at[p % 2], pesem.at[p % 2])
        cp.start()
        return cp

    def start_out(t):
        p, b = divmod(t, B)
        cp = pltpu.make_async_copy(
            xbuf.at[t % _N], o_hbm.at[b, pl.ds(p * _CH, _CH)], osem.at[t % _N])
        cp.start()
        return cp

    xin_d = [None] * _N
    out_d = [None] * _N
    pin_d = [None, None]

    pin_d[0] = start_pin(0)
    for t in range(min(_N - 1, NT)):
        xin_d[t] = start_xin(t)

    for t in range(NT):
        c = t % _N
        p, b = divmod(t, B)
        nf = t + _N - 1
        if nf < NT:
            if out_d[nf % _N] is not None:
                out_d[nf % _N].wait()
                out_d[nf % _N] = None
            xin_d[nf % _N] = start_xin(nf)
        xin_d[c].wait()
        if b == 0:
            pin_d[p % 2].wait()
            if p + 1 < NP:
                pin_d[(p + 1) % 2] = start_pin(p + 1)
        xbuf[c] = xbuf[c] + pebuf[p % 2]
        out_d[c] = start_out(t)

    for slot in range(_N):
        if out_d[slot] is not None:
            out_d[slot].wait()


def kernel(x, pe):
    B, S, D = x.shape
    return pl.pallas_call(
        _body,
        in_specs=[
            pl.BlockSpec(memory_space=pl.ANY),
            pl.BlockSpec(memory_space=pl.ANY),
        ],
        out_specs=pl.BlockSpec(memory_space=pl.ANY),
        out_shape=jax.ShapeDtypeStruct(x.shape, x.dtype),
        scratch_shapes=[
            pltpu.VMEM((_N, _CH, D), jnp.float32),
            pltpu.VMEM((2, _CH, D), jnp.float32),
            pltpu.SemaphoreType.DMA((_N,)),
            pltpu.SemaphoreType.DMA((2,)),
            pltpu.SemaphoreType.DMA((_N,)),
        ],
    )(x, pe)


# speedup vs baseline: 1.1807x; 1.1807x over previous
import jax
import jax.numpy as jnp
from jax.experimental import pallas as pl
from jax.experimental.pallas import tpu as pltpu

_N = 4
_CH = 512


def _body(x_hbm, pe_hbm, o_hbm, xbuf, obuf, pebuf, xsem, pesem, osem):
    B, S, D = x_hbm.shape
    NP = S // _CH
    NT = NP * B

    def start_xin(t):
        p, b = divmod(t, B)
        cp = pltpu.make_async_copy(
            x_hbm.at[b, pl.ds(p * _CH, _CH)], xbuf.at[t % _N],
            xsem.at[t % _N])
        cp.start()
        return cp

    def start_pin(p):
        cp = pltpu.make_async_copy(
            pe_hbm.at[pl.ds(p * _CH, _CH)], pebuf.at[p % 2], pesem.at[p % 2])
        cp.start()
        return cp

    def start_out(t):
        p, b = divmod(t, B)
        cp = pltpu.make_async_copy(
            obuf.at[t % _N], o_hbm.at[b, pl.ds(p * _CH, _CH)], osem.at[t % _N])
        cp.start()
        return cp

    xin_d = [None] * _N
    out_d = [None] * _N
    pin_d = [None, None]

    pin_d[0] = start_pin(0)
    for t in range(min(_N - 1, NT)):
        xin_d[t] = start_xin(t)

    for t in range(NT):
        c = t % _N
        p, b = divmod(t, B)
        if t + _N - 1 < NT:
            xin_d[(t + _N - 1) % _N] = start_xin(t + _N - 1)
        xin_d[c].wait()
        if b == 0:
            pin_d[p % 2].wait()
            if p + 1 < NP:
                pin_d[(p + 1) % 2] = start_pin(p + 1)
        if out_d[c] is not None:
            out_d[c].wait()
        obuf[c] = xbuf[c] + pebuf[p % 2]
        out_d[c] = start_out(t)

    for slot in range(_N):
        if out_d[slot] is not None:
            out_d[slot].wait()


def kernel(x, pe):
    B, S, D = x.shape
    return pl.pallas_call(
        _body,
        in_specs=[
            pl.BlockSpec(memory_space=pl.ANY),
            pl.BlockSpec(memory_space=pl.ANY),
        ],
        out_specs=pl.BlockSpec(memory_space=pl.ANY),
        out_shape=jax.ShapeDtypeStruct(x.shape, x.dtype),
        scratch_shapes=[
            pltpu.VMEM((_N, _CH, D), jnp.float32),
            pltpu.VMEM((_N, _CH, D), jnp.float32),
            pltpu.VMEM((2, _CH, D), jnp.float32),
            pltpu.SemaphoreType.DMA((_N,)),
            pltpu.SemaphoreType.DMA((2,)),
            pltpu.SemaphoreType.DMA((_N,)),
        ],
    )(x, pe)


# speedup vs baseline: 1.1989x; 1.0154x over previous
import jax
import jax.numpy as jnp
from jax.experimental import pallas as pl
from jax.experimental.pallas import tpu as pltpu


def _add_body(x_ref, pe_ref, o_ref):
    o_ref[...] = x_ref[...] + pe_ref[...]


def kernel(x, pe):
    B, S, D = x.shape
    BS = 2048
    return pl.pallas_call(
        _add_body,
        grid=(S // BS, B),
        in_specs=[
            pl.BlockSpec((1, BS, D), lambda s, b: (b, s, 0)),
            pl.BlockSpec((BS, D), lambda s, b: (s, 0)),
        ],
        out_specs=pl.BlockSpec((1, BS, D), lambda s, b: (b, s, 0)),
        out_shape=jax.ShapeDtypeStruct(x.shape, x.dtype),
        compiler_params=pltpu.CompilerParams(
            dimension_semantics=("parallel", "parallel")),
    )(x, pe)
